# baseline (speedup 1.0000x reference)
import functools

import jax
import jax.numpy as jnp
from jax import lax
from jax.experimental import pallas as pl
from jax.experimental.pallas import tpu as pltpu
from jax.experimental.pallas import tpu_sc as plsc

VOCAB = 50257
L = 16
NSLAB = 400
NC, NS = 2, 16
NW = NC * NS
NCHUNK = 1


def _prep_body(vals_ref, logv_ref, floor_ref):
    vals = vals_ref[...]
    logv_ref[...] = jnp.log(vals + 1e-40)
    k = vals.shape[-1]
    pmass = jnp.sum(vals, axis=-1, keepdims=True)
    rem = jnp.clip(1.0 - pmass, 1e-40, 1.0)
    lf = jnp.log(rem / (VOCAB - k))
    nb = floor_ref.shape[0]
    floor_ref[...] = jnp.broadcast_to(lf.reshape(nb, 8, 1), floor_ref.shape)


def _make_sc_body(rows_per_worker):
    def _sc_body(idx_hbm, logv_hbm, floor_hbm, out_hbm,
                 idx_v0, idx_v1, lv_v0, lv_v1, fl_v, row_v0, row_v1,
                 sin0, sin1, sout0, sout1):
        w = lax.axis_index("s") * NC + lax.axis_index("c")
        ngrp = (idx_hbm.shape[1] * idx_hbm.shape[2]) // L
        first = w * rows_per_worker
        pltpu.sync_copy(floor_hbm.at[first // 8], fl_v)

        ibufs, lbufs = [idx_v0, idx_v1], [lv_v0, lv_v1]
        rbufs = [row_v0, row_v1]
        sins, souts = [sin0, sin1], [sout0, sout1]
        pend_in = [None, None]
        pend_out = [None, None]

        def start_in(j):
            row = first + j
            p0 = pltpu.async_copy(idx_hbm.at[row], ibufs[j % 2], sins[j % 2])
            p1 = pltpu.async_copy(logv_hbm.at[row], lbufs[j % 2], sins[j % 2])
            pend_in[j % 2] = (p0, p1)

        start_in(0)
        for j in range(rows_per_worker):
            row = first + j
            rv, so = rbufs[j % 2], souts[j % 2]
            iv_b, lv_b = ibufs[j % 2], lbufs[j % 2]
            if pend_out[j % 2] is not None:
                pend_out[j % 2].wait()
            f = fl_v[(first % 8) + j, pl.ds(0, L)]

            @pl.loop(0, NSLAB, unroll=4)
            def _fill(q, rv=rv, f=f):
                for t in range(128 // L):
                    rv[q, pl.ds(t * L, L)] = f

            for p in pend_in[j % 2]:
                p.wait()
            if j + 1 < rows_per_worker:
                start_in(j + 1)

            @pl.loop(0, ngrp, unroll=4)
            def _scat(g, rv=rv, iv_b=iv_b, lv_b=lv_b):
                q = g >> 3
                lo = (g & 7) * L
                iv = iv_b[q, pl.ds(lo, L)]
                vv = lv_b[q, pl.ds(lo, L)]
                plsc.store_scatter(rv, [iv >> 7, iv & 127], vv)

            pend_out[j % 2] = pltpu.async_copy(rv, out_hbm.at[row], so)
        for p in pend_out:
            if p is not None:
                p.wait()
    return _sc_body


def _relayout_body(in_ref, *rest):
    out_ref = rest[-1]
    nb = out_ref.shape[0]
    tl = VOCAB // 128
    rem = VOCAB - tl * 128
    for bb in range(nb):
        for t in range(tl):
            out_ref[bb, :, t * 128:(t + 1) * 128] = \
                in_ref[bb * 8:(bb + 1) * 8, t, :]
        out_ref[bb, :, tl * 128:] = in_ref[bb * 8:(bb + 1) * 8, tl, :rem]


def kernel(topk_values, topk_indices, vocab_size):
    b, s, k = topk_values.shape
    r = b * s
    rows_chunk = r // NCHUNK
    rpw = rows_chunk // NW
    batches_chunk = b // NCHUNK

    vals4 = topk_values.reshape(NCHUNK, rows_chunk, k)
    idx4 = topk_indices.reshape(NCHUNK, rows_chunk, k // 128, 128)

    mesh = plsc.VectorSubcoreMesh(
        core_axis_name="c", subcore_axis_name="s",
        num_cores=NC, num_subcores=NS)

    prep = pl.pallas_call(
        _prep_body,
        grid=(2,),
        in_specs=[pl.BlockSpec((rows_chunk // 2, k), lambda i: (i, 0))],
        out_specs=[
            pl.BlockSpec((rows_chunk // 2, k), lambda i: (i, 0)),
            pl.BlockSpec((batches_chunk // 2, s, 128), lambda i: (i, 0, 0)),
        ],
        out_shape=[
            jax.ShapeDtypeStruct((rows_chunk, k), jnp.float32),
            jax.ShapeDtypeStruct((batches_chunk, s, 128), jnp.float32),
        ],
    )

    sc = functools.partial(
        pl.kernel,
        out_type=jax.ShapeDtypeStruct((rows_chunk, NSLAB, 128), jnp.float32),
        mesh=mesh,
        compiler_params=pltpu.CompilerParams(
            needs_layout_passes=False, use_tc_tiling_on_sc=True),
        scratch_types=[
            pltpu.VMEM((k // 128, 128), jnp.int32),
            pltpu.VMEM((k // 128, 128), jnp.int32),
            pltpu.VMEM((k // 128, 128), jnp.float32),
            pltpu.VMEM((k // 128, 128), jnp.float32),
            pltpu.VMEM((s, 128), jnp.float32),
            pltpu.VMEM((NSLAB, 128), jnp.float32),
            pltpu.VMEM((NSLAB, 128), jnp.float32),
            pltpu.SemaphoreType.DMA,
            pltpu.SemaphoreType.DMA,
            pltpu.SemaphoreType.DMA,
            pltpu.SemaphoreType.DMA,
        ],
    )(_make_sc_body(rpw))

    staged = []
    for c in range(NCHUNK):
        logv_c, floor_c = prep(vals4[c])
        idx3_c = idx4[c].reshape(rows_chunk, k // 128, 128)
        logv3_c = logv_c.reshape(rows_chunk, k // 128, 128)
        staged.append(sc(idx3_c, logv3_c, floor_c))

    out = None
    for c in range(NCHUNK):
        rb = 2
        in_specs = [pl.BlockSpec((rb * s, NSLAB, 128), lambda i: (i, 0, 0))]
        operands = [staged[c]]
        aliases = {}
        if out is not None:
            in_specs.append(pl.BlockSpec(memory_space=pl.ANY))
            operands.append(out)
            aliases = {1: 0}
        out = pl.pallas_call(
            _relayout_body,
            grid=(batches_chunk // rb,),
            in_specs=in_specs,
            out_specs=pl.BlockSpec(
                (rb, s, VOCAB),
                lambda i, c=c: (c * batches_chunk // rb + i, 0, 0)),
            out_shape=jax.ShapeDtypeStruct((b, s, VOCAB), jnp.float32),
            input_output_aliases=aliases,
        )(*operands)
    return out


# speedup vs baseline: 1.0513x; 1.0513x over previous
import functools

import jax
import jax.numpy as jnp
from jax import lax
from jax.experimental import pallas as pl
from jax.experimental.pallas import tpu as pltpu
from jax.experimental.pallas import tpu_sc as plsc

VOCAB = 50257
L = 16
NSLAB = 400
NC, NS = 2, 16
NW = NC * NS
NCHUNK = 2


def _prep_body(vals_ref, logv_ref, floor_ref):
    vals = vals_ref[...]
    logv_ref[...] = jnp.log(vals + 1e-40)
    k = vals.shape[-1]
    pmass = jnp.sum(vals, axis=-1, keepdims=True)
    rem = jnp.clip(1.0 - pmass, 1e-40, 1.0)
    lf = jnp.log(rem / (VOCAB - k))
    nb = floor_ref.shape[0]
    floor_ref[...] = jnp.broadcast_to(lf.reshape(nb, 8, 1), floor_ref.shape)


def _make_sc_body(rows_per_worker):
    def _sc_body(idx_hbm, logv_hbm, floor_hbm, out_hbm,
                 idx_v0, idx_v1, lv_v0, lv_v1, fl_v, row_v0, row_v1,
                 sin0, sin1, sout0, sout1):
        w = lax.axis_index("s") * NC + lax.axis_index("c")
        ngrp = (idx_hbm.shape[1] * idx_hbm.shape[2]) // L
        first = w * rows_per_worker
        pltpu.sync_copy(floor_hbm.at[first // 8], fl_v)

        ibufs, lbufs = [idx_v0, idx_v1], [lv_v0, lv_v1]
        rbufs = [row_v0, row_v1]
        sins, souts = [sin0, sin1], [sout0, sout1]
        pend_in = [None, None]
        pend_out = [None, None]

        def start_in(j):
            row = first + j
            p0 = pltpu.async_copy(idx_hbm.at[row], ibufs[j % 2], sins[j % 2])
            p1 = pltpu.async_copy(logv_hbm.at[row], lbufs[j % 2], sins[j % 2])
            pend_in[j % 2] = (p0, p1)

        start_in(0)
        for j in range(rows_per_worker):
            row = first + j
            rv, so = rbufs[j % 2], souts[j % 2]
            iv_b, lv_b = ibufs[j % 2], lbufs[j % 2]
            if pend_out[j % 2] is not None:
                pend_out[j % 2].wait()
            f = fl_v[(first % 8) + j, pl.ds(0, L)]

            @pl.loop(0, NSLAB, unroll=4)
            def _fill(q, rv=rv, f=f):
                for t in range(128 // L):
                    rv[q, pl.ds(t * L, L)] = f

            for p in pend_in[j % 2]:
                p.wait()
            if j + 1 < rows_per_worker:
                start_in(j + 1)

            @pl.loop(0, ngrp, unroll=4)
            def _scat(g, rv=rv, iv_b=iv_b, lv_b=lv_b):
                q = g >> 3
                lo = (g & 7) * L
                iv = iv_b[q, pl.ds(lo, L)]
                vv = lv_b[q, pl.ds(lo, L)]
                plsc.store_scatter(rv, [iv >> 7, iv & 127], vv)

            pend_out[j % 2] = pltpu.async_copy(rv, out_hbm.at[row], so)
        for p in pend_out:
            if p is not None:
                p.wait()
    return _sc_body


def _relayout_body(in_ref, *rest):
    out_ref = rest[-1]
    nb = out_ref.shape[0]
    tl = VOCAB // 128
    rem = VOCAB - tl * 128
    for bb in range(nb):
        for t in range(tl):
            out_ref[bb, :, t * 128:(t + 1) * 128] = \
                in_ref[bb * 8:(bb + 1) * 8, t, :]
        out_ref[bb, :, tl * 128:] = in_ref[bb * 8:(bb + 1) * 8, tl, :rem]


def kernel(topk_values, topk_indices, vocab_size):
    b, s, k = topk_values.shape
    r = b * s
    rows_chunk = r // NCHUNK
    rpw = rows_chunk // NW
    batches_chunk = b // NCHUNK

    vals4 = topk_values.reshape(NCHUNK, rows_chunk, k)
    idx4 = topk_indices.reshape(NCHUNK, rows_chunk, k // 128, 128)

    mesh = plsc.VectorSubcoreMesh(
        core_axis_name="c", subcore_axis_name="s",
        num_cores=NC, num_subcores=NS)

    prep = pl.pallas_call(
        _prep_body,
        grid=(2,),
        in_specs=[pl.BlockSpec((rows_chunk // 2, k), lambda i: (i, 0))],
        out_specs=[
            pl.BlockSpec((rows_chunk // 2, k), lambda i: (i, 0)),
            pl.BlockSpec((batches_chunk // 2, s, 128), lambda i: (i, 0, 0)),
        ],
        out_shape=[
            jax.ShapeDtypeStruct((rows_chunk, k), jnp.float32),
            jax.ShapeDtypeStruct((batches_chunk, s, 128), jnp.float32),
        ],
    )

    sc = functools.partial(
        pl.kernel,
        out_type=jax.ShapeDtypeStruct((rows_chunk, NSLAB, 128), jnp.float32),
        mesh=mesh,
        compiler_params=pltpu.CompilerParams(
            needs_layout_passes=False, use_tc_tiling_on_sc=True),
        scratch_types=[
            pltpu.VMEM((k // 128, 128), jnp.int32),
            pltpu.VMEM((k // 128, 128), jnp.int32),
            pltpu.VMEM((k // 128, 128), jnp.float32),
            pltpu.VMEM((k // 128, 128), jnp.float32),
            pltpu.VMEM((s, 128), jnp.float32),
            pltpu.VMEM((NSLAB, 128), jnp.float32),
            pltpu.VMEM((NSLAB, 128), jnp.float32),
            pltpu.SemaphoreType.DMA,
            pltpu.SemaphoreType.DMA,
            pltpu.SemaphoreType.DMA,
            pltpu.SemaphoreType.DMA,
        ],
    )(_make_sc_body(rpw))

    staged = []
    for c in range(NCHUNK):
        logv_c, floor_c = prep(vals4[c])
        idx3_c = idx4[c].reshape(rows_chunk, k // 128, 128)
        logv3_c = logv_c.reshape(rows_chunk, k // 128, 128)
        staged.append(sc(idx3_c, logv3_c, floor_c))

    out = None
    for c in range(NCHUNK):
        rb = 2
        in_specs = [pl.BlockSpec((rb * s, NSLAB, 128), lambda i: (i, 0, 0))]
        operands = [staged[c]]
        aliases = {}
        if out is not None:
            in_specs.append(pl.BlockSpec(memory_space=pl.ANY))
            operands.append(out)
            aliases = {1: 0}
        out = pl.pallas_call(
            _relayout_body,
            grid=(batches_chunk // rb,),
            in_specs=in_specs,
            out_specs=pl.BlockSpec(
                (rb, s, VOCAB),
                lambda i, c=c: (c * batches_chunk // rb + i, 0, 0)),
            out_shape=jax.ShapeDtypeStruct((b, s, VOCAB), jnp.float32),
            input_output_aliases=aliases,
        )(*operands)
    return out


# speedup vs baseline: 1.0777x; 1.0251x over previous
import functools

import jax
import jax.numpy as jnp
from jax import lax
from jax.experimental import pallas as pl
from jax.experimental.pallas import tpu as pltpu
from jax.experimental.pallas import tpu_sc as plsc

VOCAB = 50257
L = 16
NSLAB = 400
NC, NS = 2, 16
NW = NC * NS
NCHUNK = 2


def _prep_body(vals_ref, logv_ref, floor_ref):
    vals = vals_ref[...]
    logv_ref[...] = jnp.log(vals + 1e-40)
    k = vals.shape[-1]
    pmass = jnp.sum(vals, axis=-1, keepdims=True)
    rem = jnp.clip(1.0 - pmass, 1e-40, 1.0)
    lf = jnp.log(rem / (VOCAB - k))
    nb = floor_ref.shape[0]
    floor_ref[...] = jnp.broadcast_to(lf.reshape(nb, 8, 1), floor_ref.shape)


def _make_sc_body(rows_per_worker):
    def _sc_body(idx_hbm, logv_hbm, floor_hbm, out_hbm,
                 idx_v0, idx_v1, lv_v0, lv_v1, fl_v, row_v0, row_v1,
                 sin0, sin1, sout0, sout1):
        w = lax.axis_index("s") * NC + lax.axis_index("c")
        ngrp = (idx_hbm.shape[1] * idx_hbm.shape[2]) // L
        first = w * rows_per_worker
        pltpu.sync_copy(floor_hbm.at[first // 8], fl_v)

        ibufs, lbufs = [idx_v0, idx_v1], [lv_v0, lv_v1]
        rbufs = [row_v0, row_v1]
        sins, souts = [sin0, sin1], [sout0, sout1]
        pend_in = [None, None]
        pend_out = [None, None]

        def start_in(j):
            row = first + j
            p0 = pltpu.async_copy(idx_hbm.at[row], ibufs[j % 2], sins[j % 2])
            p1 = pltpu.async_copy(logv_hbm.at[row], lbufs[j % 2], sins[j % 2])
            pend_in[j % 2] = (p0, p1)

        start_in(0)
        for j in range(rows_per_worker):
            row = first + j
            rv, so = rbufs[j % 2], souts[j % 2]
            iv_b, lv_b = ibufs[j % 2], lbufs[j % 2]
            if pend_out[j % 2] is not None:
                pend_out[j % 2].wait()
            f = fl_v[(first % 8) + j, pl.ds(0, L)]

            @pl.loop(0, NSLAB, unroll=4)
            def _fill(q, rv=rv, f=f):
                for t in range(128 // L):
                    rv[q, pl.ds(t * L, L)] = f

            for p in pend_in[j % 2]:
                p.wait()
            if j + 1 < rows_per_worker:
                start_in(j + 1)

            @pl.loop(0, ngrp, unroll=4)
            def _scat(g, rv=rv, iv_b=iv_b, lv_b=lv_b):
                q = g >> 3
                lo = (g & 7) * L
                iv = iv_b[q, pl.ds(lo, L)]
                vv = lv_b[q, pl.ds(lo, L)]
                plsc.store_scatter(rv, [iv >> 7, iv & 127], vv)

            pend_out[j % 2] = pltpu.async_copy(rv, out_hbm.at[row], so)
        for p in pend_out:
            if p is not None:
                p.wait()
    return _sc_body


def _relayout_body(in_ref, *rest):
    out_ref = rest[-1]
    nb = out_ref.shape[0]
    tl = VOCAB // 128
    rem = VOCAB - tl * 128
    for bb in range(nb):
        for t in range(tl):
            out_ref[bb, :, t * 128:(t + 1) * 128] = \
                in_ref[bb * 8:(bb + 1) * 8, t, :]
        out_ref[bb, :, tl * 128:] = in_ref[bb * 8:(bb + 1) * 8, tl, :rem]


def kernel(topk_values, topk_indices, vocab_size):
    b, s, k = topk_values.shape
    r = b * s
    rows_chunk = r // NCHUNK
    rpw = rows_chunk // NW
    batches_chunk = b // NCHUNK

    vals4 = topk_values.reshape(NCHUNK, rows_chunk, k)
    idx4 = topk_indices.reshape(NCHUNK, rows_chunk, k // 128, 128)

    mesh = plsc.VectorSubcoreMesh(
        core_axis_name="c", subcore_axis_name="s",
        num_cores=NC, num_subcores=NS)

    prep = pl.pallas_call(
        _prep_body,
        grid=(2,),
        in_specs=[pl.BlockSpec((rows_chunk // 2, k), lambda i: (i, 0))],
        out_specs=[
            pl.BlockSpec((rows_chunk // 2, k), lambda i: (i, 0)),
            pl.BlockSpec((batches_chunk // 2, s, 128), lambda i: (i, 0, 0)),
        ],
        out_shape=[
            jax.ShapeDtypeStruct((rows_chunk, k), jnp.float32),
            jax.ShapeDtypeStruct((batches_chunk, s, 128), jnp.float32),
        ],
    )

    sc = functools.partial(
        pl.kernel,
        out_type=jax.ShapeDtypeStruct((rows_chunk, NSLAB, 128), jnp.float32),
        mesh=mesh,
        compiler_params=pltpu.CompilerParams(
            needs_layout_passes=False, use_tc_tiling_on_sc=True),
        scratch_types=[
            pltpu.VMEM((k // 128, 128), jnp.int32),
            pltpu.VMEM((k // 128, 128), jnp.int32),
            pltpu.VMEM((k // 128, 128), jnp.float32),
            pltpu.VMEM((k // 128, 128), jnp.float32),
            pltpu.VMEM((s, 128), jnp.float32),
            pltpu.VMEM((NSLAB, 128), jnp.float32),
            pltpu.VMEM((NSLAB, 128), jnp.float32),
            pltpu.SemaphoreType.DMA,
            pltpu.SemaphoreType.DMA,
            pltpu.SemaphoreType.DMA,
            pltpu.SemaphoreType.DMA,
        ],
    )(_make_sc_body(rpw))

    staged = []
    for c in range(NCHUNK):
        logv_c, floor_c = prep(vals4[c])
        idx3_c = idx4[c].reshape(rows_chunk, k // 128, 128)
        logv3_c = logv_c.reshape(rows_chunk, k // 128, 128)
        staged.append(sc(idx3_c, logv3_c, floor_c))

    out = None
    for c in range(NCHUNK):
        rb = 4
        in_specs = [pl.BlockSpec((rb * s, NSLAB, 128), lambda i: (i, 0, 0))]
        operands = [staged[c]]
        aliases = {}
        if out is not None:
            in_specs.append(pl.BlockSpec(memory_space=pl.ANY))
            operands.append(out)
            aliases = {1: 0}
        out = pl.pallas_call(
            _relayout_body,
            grid=(batches_chunk // rb,),
            in_specs=in_specs,
            out_specs=pl.BlockSpec(
                (rb, s, VOCAB),
                lambda i, c=c: (c * batches_chunk // rb + i, 0, 0)),
            out_shape=jax.ShapeDtypeStruct((b, s, VOCAB), jnp.float32),
            input_output_aliases=aliases,
        )(*operands)
    return out
